# Initial kernel scaffold; baseline (speedup 1.0000x reference)
import jax
import jax.numpy as jnp
from jax.experimental import pallas as pl


def kernel(z_real, z_imag, cb_real, cb_imag):
    raise NotImplementedError("write your pallas kernel here")


# speedup vs baseline: 1.2737x; 1.2737x over previous
import functools

import jax
import jax.numpy as jnp
from jax.experimental import pallas as pl
from jax.experimental.pallas import tpu as pltpu

B, N, D, K = 32, 576, 256, 8192
M = B * N

MB = 512
KB = 1024


def _vq_kernel(zr_ref, zi_ref, cbr_ref, cbi_ref, out_ref, best_ref):
    k = pl.program_id(1)

    zr = zr_ref[...]
    zi = zi_ref[...]
    cbr = cbr_ref[...]
    cbi = cbi_ref[...]

    dot = functools.partial(
        jax.lax.dot_general,
        dimension_numbers=(((1,), (1,)), ((), ())),
        preferred_element_type=jnp.float32,
    )
    re = dot(zr, cbr) + dot(zi, cbi)
    im = dot(zr, cbi) - dot(zi, cbr)
    mag2 = jnp.sqrt(re * re + im * im)

    loc_max = jnp.max(mag2, axis=1)
    iota = jax.lax.broadcasted_iota(jnp.int32, (MB, KB), 1)
    masked = jnp.where(mag2 == loc_max[:, None], iota, K)
    loc_arg = jnp.min(masked, axis=1) + k * KB

    @pl.when(k == 0)
    def _init():
        best_ref[...] = loc_max
        out_ref[0, 0, :] = loc_arg

    @pl.when(k > 0)
    def _update():
        prev = best_ref[...]
        upd = loc_max > prev
        out_ref[0, 0, :] = jnp.where(upd, loc_arg, out_ref[0, 0, :])
        best_ref[...] = jnp.maximum(prev, loc_max)


def kernel(z_real, z_imag, cb_real, cb_imag):
    zr = z_real.reshape(M, D).astype(jnp.bfloat16)
    zi = z_imag.reshape(M, D).astype(jnp.bfloat16)
    cb_real = cb_real.astype(jnp.bfloat16)
    cb_imag = cb_imag.astype(jnp.bfloat16)

    grid = (M // MB, K // KB)
    out = pl.pallas_call(
        _vq_kernel,
        grid=grid,
        in_specs=[
            pl.BlockSpec((MB, D), lambda m, k: (m, 0)),
            pl.BlockSpec((MB, D), lambda m, k: (m, 0)),
            pl.BlockSpec((KB, D), lambda m, k: (k, 0)),
            pl.BlockSpec((KB, D), lambda m, k: (k, 0)),
        ],
        out_specs=pl.BlockSpec((1, 1, MB), lambda m, k: (m, 0, 0)),
        out_shape=jax.ShapeDtypeStruct((M // MB, 1, MB), jnp.int32),
        scratch_shapes=[pltpu.VMEM((MB,), jnp.float32)],
    )(zr, zi, cb_real, cb_imag)
    return out.reshape(B, N)


# speedup vs baseline: 1.8264x; 1.4340x over previous
import functools

import jax
import jax.numpy as jnp
from jax.experimental import pallas as pl
from jax.experimental.pallas import tpu as pltpu

B, N, D, K = 32, 576, 256, 8192
M = B * N

MB = 512
KB = 2048


def _vq_kernel(zr_ref, zi_ref, cbr_ref, cbi_ref, out_ref, best_ref):
    k = pl.program_id(1)

    zr = zr_ref[...]
    zi = zi_ref[...]
    cbr = cbr_ref[...]
    cbi = cbi_ref[...]

    dot = functools.partial(
        jax.lax.dot_general,
        dimension_numbers=(((1,), (1,)), ((), ())),
        preferred_element_type=jnp.float32,
    )
    re = dot(zr, cbr) + dot(zi, cbi)
    im = dot(zr, cbi) - dot(zi, cbr)
    mag2 = re * re + im * im

    loc_max = jnp.max(mag2, axis=1)
    iota = jax.lax.broadcasted_iota(jnp.int32, (MB, KB), 1)
    masked = jnp.where(mag2 == loc_max[:, None], iota, K)
    loc_arg = jnp.min(masked, axis=1) + k * KB

    @pl.when(k == 0)
    def _init():
        best_ref[...] = loc_max
        out_ref[0, 0, :] = loc_arg

    @pl.when(k > 0)
    def _update():
        prev = best_ref[...]
        upd = loc_max > prev
        out_ref[0, 0, :] = jnp.where(upd, loc_arg, out_ref[0, 0, :])
        best_ref[...] = jnp.maximum(prev, loc_max)


def kernel(z_real, z_imag, cb_real, cb_imag):
    zr = z_real.reshape(M, D).astype(jnp.bfloat16)
    zi = z_imag.reshape(M, D).astype(jnp.bfloat16)
    cb_real = cb_real.astype(jnp.bfloat16)
    cb_imag = cb_imag.astype(jnp.bfloat16)

    grid = (M // MB, K // KB)
    out = pl.pallas_call(
        _vq_kernel,
        grid=grid,
        in_specs=[
            pl.BlockSpec((MB, D), lambda m, k: (m, 0)),
            pl.BlockSpec((MB, D), lambda m, k: (m, 0)),
            pl.BlockSpec((KB, D), lambda m, k: (k, 0)),
            pl.BlockSpec((KB, D), lambda m, k: (k, 0)),
        ],
        out_specs=pl.BlockSpec((1, 1, MB), lambda m, k: (m, 0, 0)),
        out_shape=jax.ShapeDtypeStruct((M // MB, 1, MB), jnp.int32),
        scratch_shapes=[pltpu.VMEM((MB,), jnp.float32)],
    )(zr, zi, cb_real, cb_imag)
    return out.reshape(B, N)


# speedup vs baseline: 1.9621x; 1.0743x over previous
import functools

import jax
import jax.numpy as jnp
from jax.experimental import pallas as pl
from jax.experimental.pallas import tpu as pltpu

B, N, D, K = 32, 576, 256, 8192
M = B * N

MB = 1024
KB = 2048


def _vq_kernel(zr_ref, zi_ref, cbr_ref, cbi_ref, out_ref, best_ref):
    k = pl.program_id(1)

    zr = zr_ref[...]
    zi = zi_ref[...]
    cbr = cbr_ref[...]
    cbi = cbi_ref[...]

    dot = functools.partial(
        jax.lax.dot_general,
        dimension_numbers=(((1,), (1,)), ((), ())),
        preferred_element_type=jnp.float32,
    )
    re = dot(zr, cbr) + dot(zi, cbi)
    im = dot(zr, cbi) - dot(zi, cbr)
    mag2 = re * re + im * im

    loc_max = jnp.max(mag2, axis=1)
    iota = jax.lax.broadcasted_iota(jnp.int32, (MB, KB), 1)
    masked = jnp.where(mag2 == loc_max[:, None], iota, K)
    loc_arg = jnp.min(masked, axis=1) + k * KB

    @pl.when(k == 0)
    def _init():
        best_ref[...] = loc_max
        out_ref[0, 0, :] = loc_arg

    @pl.when(k > 0)
    def _update():
        prev = best_ref[...]
        upd = loc_max > prev
        out_ref[0, 0, :] = jnp.where(upd, loc_arg, out_ref[0, 0, :])
        best_ref[...] = jnp.maximum(prev, loc_max)


def kernel(z_real, z_imag, cb_real, cb_imag):
    zr = z_real.reshape(M, D).astype(jnp.bfloat16)
    zi = z_imag.reshape(M, D).astype(jnp.bfloat16)
    cb_real = cb_real.astype(jnp.bfloat16)
    cb_imag = cb_imag.astype(jnp.bfloat16)

    grid = (M // MB, K // KB)
    out = pl.pallas_call(
        _vq_kernel,
        grid=grid,
        in_specs=[
            pl.BlockSpec((MB, D), lambda m, k: (m, 0)),
            pl.BlockSpec((MB, D), lambda m, k: (m, 0)),
            pl.BlockSpec((KB, D), lambda m, k: (k, 0)),
            pl.BlockSpec((KB, D), lambda m, k: (k, 0)),
        ],
        out_specs=pl.BlockSpec((1, 1, MB), lambda m, k: (m, 0, 0)),
        out_shape=jax.ShapeDtypeStruct((M // MB, 1, MB), jnp.int32),
        scratch_shapes=[pltpu.VMEM((MB,), jnp.float32)],
    )(zr, zi, cb_real, cb_imag)
    return out.reshape(B, N)
